# baseline (speedup 1.0000x reference)
import functools

import jax
import jax.numpy as jnp
from jax import lax
from jax.experimental import pallas as pl
from jax.experimental.pallas import tpu as pltpu
from jax.experimental.pallas import tpu_sc as plsc

_THRESHOLD = 0.7
_KEEP_NUM = 32768
_N = 131072
_C = 256

_BR = 2048
_NB = _N // _BR
_R2 = 1024
_C2 = _N // _R2

_NC = 2
_NS = 16
_NW = _NC * _NS
_CHUNK = _N // _NW
_CROWS = _CHUNK // 128


def _logz_body(x_ref, logz_ref):
    x = x_ref[...]
    m = jnp.max(x, axis=1, keepdims=True)
    e = jnp.exp(x - m)
    s = jnp.sum(e, axis=1)
    logz_ref[0, 0, :] = m[:, 0] + jnp.log(s)


def _sc_pick_body(outflat_hbm, tgt_hbm, picked_hbm, idx_v, rows_v, sem):
    wid = lax.axis_index("s") * _NC + lax.axis_index("c")
    base = wid * _CHUNK
    pltpu.sync_copy(tgt_hbm.at[wid], idx_v)

    def body(j, carry):
        t16 = idx_v[pl.ds(j * 16, 16)]
        g = base + j * 16
        rowid = g + lax.iota(jnp.int32, 16)
        idx_v[pl.ds(j * 16, 16)] = rowid * _C + t16
        return carry

    lax.fori_loop(0, _CHUNK // 16, body, 0)
    pltpu.async_copy(outflat_hbm.at[idx_v], rows_v, sem).wait()
    pltpu.sync_copy(rows_v, picked_hbm.at[wid])


def _select_body(logz_ref, picked_ref, out_ref, keys_ref, x_ref):
    _SIGN = jnp.int32(-2 ** 31)
    x = logz_ref[...] - picked_ref[...]
    x_ref[...] = x
    bits = lax.bitcast_convert_type(x, jnp.int32)
    ikey = jnp.where(bits >= 0, bits,
                     jnp.bitwise_xor(jnp.bitwise_not(bits), _SIGN))
    keys_ref[...] = ikey

    k1 = jnp.int32(_KEEP_NUM)
    k2 = jnp.int32(_KEEP_NUM + 1)

    def body(b, carry):
        p1, p2 = carry
        bit = jnp.left_shift(jnp.int32(1), 31 - b)
        c1 = jnp.bitwise_or(p1, bit)
        c2 = jnp.bitwise_or(p2, bit)
        k = keys_ref[...]
        cnt1 = jnp.sum((k >= jnp.bitwise_xor(c1, _SIGN)).astype(jnp.int32))
        cnt2 = jnp.sum((k >= jnp.bitwise_xor(c2, _SIGN)).astype(jnp.int32))
        p1 = jnp.where(cnt1 >= k1, c1, p1)
        p2 = jnp.where(cnt2 >= k2, c2, p2)
        return p1, p2

    p1, p2 = lax.fori_loop(0, 32, body, (jnp.int32(0), jnp.int32(0)))
    ikey1 = jnp.bitwise_xor(p1, _SIGN)
    ikey2 = jnp.bitwise_xor(p2, _SIGN)

    k = keys_ref[...]
    x = x_ref[...]
    v1 = jnp.max(jnp.where(k == ikey1, x, -jnp.inf))
    v2 = jnp.max(jnp.where(k == ikey2, x, -jnp.inf))

    gt1 = k > ikey1
    cnt_top = jnp.sum(gt1.astype(jnp.float32))
    sum_top = jnp.sum(jnp.where(gt1, x, 0.0))
    branch_b = (sum_top + v1 * (jnp.float32(_KEEP_NUM) - cnt_top)) \
        / jnp.float32(_KEEP_NUM)

    m7 = x > jnp.float32(_THRESHOLD)
    sum7 = jnp.sum(jnp.where(m7, x, 0.0))
    cnt7 = jnp.maximum(jnp.sum(m7.astype(jnp.float32)), 1.0)
    branch_a = sum7 / cnt7

    res = jnp.where(v2 > jnp.float32(_THRESHOLD), branch_a, branch_b)
    out_ref[...] = jnp.broadcast_to(res, (1, 1))


@functools.partial(
    pl.kernel,
    mesh=plsc.VectorSubcoreMesh(core_axis_name="c", subcore_axis_name="s"),
    out_type=jax.ShapeDtypeStruct((_NW, _CHUNK), jnp.float32),
    scratch_types=[
        pltpu.VMEM((_CHUNK,), jnp.int32),
        pltpu.VMEM((_CHUNK,), jnp.float32),
        pltpu.SemaphoreType.DMA,
    ],
)
def _sc_pick(outflat_hbm, tgt_hbm, picked_hbm, idx_v, rows_v, sem):
    _sc_pick_body(outflat_hbm, tgt_hbm, picked_hbm, idx_v, rows_v, sem)


def kernel(output, target):
    picked = _sc_pick(output.reshape(_N * _C),
                      target.reshape(_NW, _CHUNK))

    logz = pl.pallas_call(
        _logz_body,
        grid=(_NB,),
        in_specs=[pl.BlockSpec((_BR, _C), lambda i: (i, 0))],
        out_specs=pl.BlockSpec((1, 1, _BR), lambda i: (i, 0, 0)),
        out_shape=jax.ShapeDtypeStruct((_NB, 1, _BR), jnp.float32),
    )(output)

    res = pl.pallas_call(
        _select_body,
        out_shape=jax.ShapeDtypeStruct((1, 1), jnp.float32),
        scratch_shapes=[pltpu.VMEM((_R2, _C2), jnp.int32),
                        pltpu.VMEM((_R2, _C2), jnp.float32)],
    )(logz.reshape(_R2, _C2), picked.reshape(_R2, _C2))
    return res[0, 0]


# speedup vs baseline: 1.5130x; 1.5130x over previous
import jax
import jax.numpy as jnp
from jax.experimental import pallas as pl
from jax.experimental.pallas import tpu as pltpu

_THRESHOLD = 0.7
_KEEP_NUM = 32768
_N = 131072
_C = 256

_BR = 2048
_NB = _N // _BR


def _select(sa_ref, ss_ref, x_ref, keys_ref, out_ref):
    _SIGN = jnp.int32(-2 ** 31)
    x = sa_ref[...] + jnp.log(ss_ref[...])
    x_ref[...] = x
    bits = jax.lax.bitcast_convert_type(x, jnp.int32)
    ikey = jnp.where(bits >= 0, bits,
                     jnp.bitwise_xor(jnp.bitwise_not(bits), _SIGN))
    keys_ref[...] = ikey

    k1 = jnp.int32(_KEEP_NUM)
    k2 = jnp.int32(_KEEP_NUM + 1)

    def body(b, carry):
        p1, p2 = carry
        bit = jnp.left_shift(jnp.int32(1), 31 - b)
        c1 = jnp.bitwise_or(p1, bit)
        c2 = jnp.bitwise_or(p2, bit)
        k = keys_ref[...]
        cnt1 = jnp.sum((k >= jnp.bitwise_xor(c1, _SIGN)).astype(jnp.int32))
        cnt2 = jnp.sum((k >= jnp.bitwise_xor(c2, _SIGN)).astype(jnp.int32))
        p1 = jnp.where(cnt1 >= k1, c1, p1)
        p2 = jnp.where(cnt2 >= k2, c2, p2)
        return p1, p2

    p1, p2 = jax.lax.fori_loop(0, 32, body, (jnp.int32(0), jnp.int32(0)))
    ikey1 = jnp.bitwise_xor(p1, _SIGN)
    ikey2 = jnp.bitwise_xor(p2, _SIGN)

    k = keys_ref[...]
    x = x_ref[...]
    v1 = jnp.max(jnp.where(k == ikey1, x, -jnp.inf))
    v2 = jnp.max(jnp.where(k == ikey2, x, -jnp.inf))

    gt1 = k > ikey1
    cnt_top = jnp.sum(gt1.astype(jnp.float32))
    sum_top = jnp.sum(jnp.where(gt1, x, 0.0))
    branch_b = (sum_top + v1 * (jnp.float32(_KEEP_NUM) - cnt_top)) \
        / jnp.float32(_KEEP_NUM)

    m7 = x > jnp.float32(_THRESHOLD)
    sum7 = jnp.sum(jnp.where(m7, x, 0.0))
    cnt7 = jnp.maximum(jnp.sum(m7.astype(jnp.float32)), 1.0)
    branch_a = sum7 / cnt7

    res = jnp.where(v2 > jnp.float32(_THRESHOLD), branch_a, branch_b)
    out_ref[...] = jnp.broadcast_to(res, (1, 1))


def _body(x_ref, t_ref, out_ref, sa_ref, ss_ref, xd_ref, keys_ref):
    i = pl.program_id(0)
    x = x_ref[...]
    t = t_ref[...]
    m = jnp.max(x, axis=1, keepdims=True)
    e = jnp.exp(x - m)
    s = jnp.sum(e, axis=1, keepdims=True)
    cols = jax.lax.broadcasted_iota(jnp.int32, (_BR, _C), 1)
    picked = jnp.sum(jnp.where(cols == t, x, 0.0), axis=1, keepdims=True)
    lane = jax.lax.broadcasted_iota(jnp.int32, (_BR, _NB), 1)
    hit = lane == i
    sa_ref[...] = jnp.where(hit, m - picked, sa_ref[...])
    ss_ref[...] = jnp.where(hit, s, ss_ref[...])

    @pl.when(i == _NB - 1)
    def _():
        _select(sa_ref, ss_ref, xd_ref, keys_ref, out_ref)


def kernel(output, target):
    res = pl.pallas_call(
        _body,
        grid=(_NB,),
        in_specs=[
            pl.BlockSpec((_BR, _C), lambda i: (i, 0)),
            pl.BlockSpec((_BR, 1), lambda i: (i, 0)),
        ],
        out_specs=pl.BlockSpec((1, 1), lambda i: (0, 0)),
        out_shape=jax.ShapeDtypeStruct((1, 1), jnp.float32),
        scratch_shapes=[
            pltpu.VMEM((_BR, _NB), jnp.float32),
            pltpu.VMEM((_BR, _NB), jnp.float32),
            pltpu.VMEM((_BR, _NB), jnp.float32),
            pltpu.VMEM((_BR, _NB), jnp.int32),
        ],
    )(output, target.reshape(_N, 1))
    return res[0, 0]


# speedup vs baseline: 1.8598x; 1.2292x over previous
import jax
import jax.numpy as jnp
from jax.experimental import pallas as pl
from jax.experimental.pallas import tpu as pltpu

_THRESHOLD = 0.7
_KEEP_NUM = 32768
_N = 131072
_C = 256

_BR = 2048
_NB = _N // _BR


def _select(sa_ref, ss_ref, x_ref, keys_ref, out_ref):
    _SIGN = jnp.int32(-2 ** 31)
    x = sa_ref[...] + jnp.log(ss_ref[...])
    x_ref[...] = x
    bits = jax.lax.bitcast_convert_type(x, jnp.int32)
    ikey = jnp.where(bits >= 0, bits,
                     jnp.bitwise_xor(jnp.bitwise_not(bits), _SIGN))
    keys_ref[...] = ikey

    k1 = jnp.int32(_KEEP_NUM)
    k2 = jnp.int32(_KEEP_NUM + 1)

    def body(b, carry):
        p1, p2 = carry
        bit = jnp.left_shift(jnp.int32(1), 31 - b)
        c1 = jnp.bitwise_or(p1, bit)
        c2 = jnp.bitwise_or(p2, bit)
        k = keys_ref[...]
        cnt1 = jnp.sum((k >= jnp.bitwise_xor(c1, _SIGN)).astype(jnp.int32))
        cnt2 = jnp.sum((k >= jnp.bitwise_xor(c2, _SIGN)).astype(jnp.int32))
        p1 = jnp.where(cnt1 >= k1, c1, p1)
        p2 = jnp.where(cnt2 >= k2, c2, p2)
        return p1, p2

    p1, p2 = jax.lax.fori_loop(0, 32, body, (jnp.int32(0), jnp.int32(0)))
    ikey1 = jnp.bitwise_xor(p1, _SIGN)
    ikey2 = jnp.bitwise_xor(p2, _SIGN)

    k = keys_ref[...]
    x = x_ref[...]
    v1 = jnp.max(jnp.where(k == ikey1, x, -jnp.inf))
    v2 = jnp.max(jnp.where(k == ikey2, x, -jnp.inf))

    gt1 = k > ikey1
    cnt_top = jnp.sum(gt1.astype(jnp.float32))
    sum_top = jnp.sum(jnp.where(gt1, x, 0.0))
    branch_b = (sum_top + v1 * (jnp.float32(_KEEP_NUM) - cnt_top)) \
        / jnp.float32(_KEEP_NUM)

    m7 = x > jnp.float32(_THRESHOLD)
    sum7 = jnp.sum(jnp.where(m7, x, 0.0))
    cnt7 = jnp.maximum(jnp.sum(m7.astype(jnp.float32)), 1.0)
    branch_a = sum7 / cnt7

    res = jnp.where(v2 > jnp.float32(_THRESHOLD), branch_a, branch_b)
    out_ref[...] = jnp.broadcast_to(res, (1, 1))


def _body(x_ref, t_ref, out_ref, sa_ref, ss_ref, xd_ref, keys_ref):
    i = pl.program_id(0)
    x = x_ref[...]
    trow = t_ref[0, :, :].astype(jnp.float32)
    tcol = jax.lax.dot_general(
        trow, jnp.ones((1, 1), jnp.float32),
        (((0,), (0,)), ((), ())))
    t = tcol.astype(jnp.int32)
    m = jnp.max(x, axis=1, keepdims=True)
    e = jnp.exp(x - m)
    s = jnp.sum(e, axis=1, keepdims=True)
    cols = jax.lax.broadcasted_iota(jnp.int32, (_BR, _C), 1)
    picked = jnp.sum(jnp.where(cols == t, x, 0.0), axis=1, keepdims=True)
    lane = jax.lax.broadcasted_iota(jnp.int32, (_BR, _NB), 1)
    hit = lane == i
    sa_ref[...] = jnp.where(hit, m - picked, sa_ref[...])
    ss_ref[...] = jnp.where(hit, s, ss_ref[...])

    @pl.when(i == _NB - 1)
    def _():
        _select(sa_ref, ss_ref, xd_ref, keys_ref, out_ref)


def kernel(output, target):
    res = pl.pallas_call(
        _body,
        grid=(_NB,),
        in_specs=[
            pl.BlockSpec((_BR, _C), lambda i: (i, 0)),
            pl.BlockSpec((1, 1, _BR), lambda i: (i, 0, 0)),
        ],
        out_specs=pl.BlockSpec((1, 1), lambda i: (0, 0)),
        out_shape=jax.ShapeDtypeStruct((1, 1), jnp.float32),
        scratch_shapes=[
            pltpu.VMEM((_BR, _NB), jnp.float32),
            pltpu.VMEM((_BR, _NB), jnp.float32),
            pltpu.VMEM((_BR, _NB), jnp.float32),
            pltpu.VMEM((_BR, _NB), jnp.int32),
        ],
    )(output, target.reshape(_NB, 1, _BR))
    return res[0, 0]
